# Initial kernel scaffold; baseline (speedup 1.0000x reference)
import jax
import jax.numpy as jnp
from jax.experimental import pallas as pl


def kernel(x, edge_index, params):
    raise NotImplementedError("write your pallas kernel here")


# speedup vs baseline: 16.6078x; 16.6078x over previous
import jax
import jax.numpy as jnp
from jax import lax
from jax.experimental import pallas as pl
from jax.experimental.pallas import tpu as pltpu
from jax.experimental.pallas import tpu_sc as plsc

N = 10000
E = 320000
HID = 128
HEADS = 8
HD = 16

NC = 2
NS = 16
NW = NC * NS
C = 32
BLK = 8
NBLK = 40
CHUNKS = BLK * NBLK
EPW = C * CHUNKS
PAD_DST = N
NPAD = 10008
ZROWS = NPAD // 8
AROWS = 11264
APS = AROWS // NS

_mesh = plsc.VectorSubcoreMesh(core_axis_name="c", subcore_axis_name="s",
                               num_cores=NC, num_subcores=NS)

_GATHER_DNUMS = lax.GatherDimensionNumbers(
    offset_dims=(), collapsed_slice_dims=(0,), start_index_map=(0,))


def _lane_shuffle(v, idx):
    return lax.gather(v, idx[:, None], _GATHER_DNUMS, slice_sizes=(1,),
                      mode=lax.GatherScatterMode.PROMISE_IN_BOUNDS)


def _edge_body(kv_hbm, q_hbm, src_hbm, dst_hbm, acc_hbm,
               srcblk, dstblk, sidx, kv_v, q_v, o2,
               acc, sem1, sem2, sem3, sem4):
    c = lax.axis_index("c")
    s = lax.axis_index("s")
    w = s * NC + c

    zero16 = jnp.zeros((16,), jnp.float32)
    lanes = lax.iota(jnp.int32, 16)

    def zero_row(j, carry):
        for col in range(HID // 16):
            o2[j, pl.ds(col * 16, 16)] = zero16
        return carry
    lax.fori_loop(0, 2 * C, zero_row, 0)
    for t in range(APS // (2 * C)):
        pltpu.sync_copy(o2, acc.at[pl.ds(s * APS + t * 2 * C, 2 * C)])
    plsc.subcore_barrier()

    pltpu.sync_copy(src_hbm.at[w, 0], srcblk.at[0])
    pltpu.sync_copy(dst_hbm.at[w, 0], dstblk.at[0])
    pltpu.async_copy(kv_hbm.at[srcblk.at[0, 0]], kv_v.at[0], sem1)
    pltpu.async_copy(q_hbm.at[dstblk.at[0, 0]], q_v.at[0], sem2)

    def chunk(i, carry):
        p = i & 1
        b = lax.shift_right_logical(i, 3)
        j = i & (BLK - 1)
        pb = b & 1

        @pl.when((j == 0) & (b + 1 < NBLK))
        def _stage_next_block():
            pltpu.async_copy(src_hbm.at[w, b + 1], srcblk.at[1 - pb], sem3)
            pltpu.async_copy(dst_hbm.at[w, b + 1], dstblk.at[1 - pb], sem4)

        pltpu.make_async_copy(kv_hbm.at[pl.ds(0, C)], kv_v.at[p], sem1).wait()
        pltpu.make_async_copy(q_hbm.at[pl.ds(0, C)], q_v.at[p], sem2).wait()

        nj = (i + 1) & (BLK - 1)
        npb = lax.shift_right_logical(i + 1, 3) & 1

        @pl.when(j == BLK - 1)
        def _wait_next_block():
            @pl.when(b + 1 < NBLK)
            def _():
                pltpu.make_async_copy(src_hbm.at[w, 0], srcblk.at[0], sem3).wait()
                pltpu.make_async_copy(dst_hbm.at[w, 0], dstblk.at[0], sem4).wait()

        @pl.when(i + 1 < CHUNKS)
        def _issue_next_gather():
            pltpu.async_copy(kv_hbm.at[srcblk.at[npb, nj]], kv_v.at[1 - p], sem1)
            pltpu.async_copy(q_hbm.at[dstblk.at[npb, nj]], q_v.at[1 - p], sem2)

        for g in range(C // 16):
            dvec = dstblk[pb, j, pl.ds(g * 16, 16)]
            sidx[pl.ds(g * 16, 16)] = dvec
            sidx[pl.ds(C + g * 16, 16)] = (
                lax.shift_right_logical(dvec, 3) + NPAD)

        def edge(e, ecarry):
            zvec = jnp.zeros((16,), jnp.float32)
            for h in range(HEADS):
                kh = kv_v[p, e, pl.ds(h * HD, 16)]
                qh = q_v[p, e, pl.ds(h * HD, 16)]
                sc = kh * qh
                for sh in (8, 4, 2, 1):
                    sc = sc + _lane_shuffle(sc, lanes ^ sh)
                sv = jnp.exp(jnp.clip(sc, -5.0, 5.0))
                vh = kv_v[p, e, pl.ds(HID + h * HD, 16)]
                o2[e, pl.ds(h * HD, 16)] = vh * sv
                zvec = jnp.where(lanes == h, sv, zvec)
            g = (e // 16) * 16
            dvec = dstblk[pb, j, pl.ds(g, 16)]
            db = _lane_shuffle(dvec, jnp.full((16,), e - g, jnp.int32))
            dmod = db & 7
            for off in range(8):
                m = 1.0 - jnp.minimum(jnp.abs(dmod - off), 1).astype(jnp.float32)
                o2[C + e, pl.ds(off * 16, 16)] = zvec * m
            return ecarry
        lax.fori_loop(0, C, edge, 0)

        pltpu.sync_copy(o2, acc.at[sidx], add=True)
        return carry
    lax.fori_loop(0, CHUNKS, chunk, 0)

    plsc.subcore_barrier()
    pltpu.sync_copy(acc.at[pl.ds(s * APS, APS)], acc_hbm.at[c, pl.ds(s * APS, APS)])


_edge_attn = pl.kernel(
    _edge_body,
    out_type=jax.ShapeDtypeStruct((NC, AROWS, HID), jnp.float32),
    mesh=_mesh,
    scratch_types=[
        pltpu.VMEM((2, BLK, C), jnp.int32),
        pltpu.VMEM((2, BLK, C), jnp.int32),
        pltpu.VMEM((2 * C,), jnp.int32),
        pltpu.VMEM((2, C, 2 * HID), jnp.float32),
        pltpu.VMEM((2, C, HID), jnp.float32),
        pltpu.VMEM((2 * C, HID), jnp.float32),
        pltpu.VMEM_SHARED((AROWS, HID), jnp.float32),
        pltpu.SemaphoreType.DMA,
        pltpu.SemaphoreType.DMA,
        pltpu.SemaphoreType.DMA,
        pltpu.SemaphoreType.DMA,
    ],
)


def _embed_body(x_ref, w_ref, o_ref):
    o_ref[...] = jnp.dot(x_ref[...], w_ref[...], preferred_element_type=jnp.float32)


def _qkv_body(h_ref, wkv_ref, bkv_ref, wq_ref, bq_ref, kv_ref, q_ref):
    hh = h_ref[...]
    kv_ref[...] = jnp.dot(hh, wkv_ref[...], preferred_element_type=jnp.float32) + bkv_ref[...]
    q_ref[...] = (jnp.dot(hh, wq_ref[...], preferred_element_type=jnp.float32)
                  + bq_ref[...]) * (HD ** -0.5)


def _ln(hh, g, b):
    mu = jnp.mean(hh, axis=-1, keepdims=True)
    var = jnp.mean((hh - mu) ** 2, axis=-1, keepdims=True)
    return (hh - mu) / jnp.sqrt(var + 1e-5) * g + b


def _post_body(h_ref, acc_ref, wo_ref, bo_ref, g1_ref, b1_ref,
               w1_ref, b1f_ref, w2_ref, b2f_ref, g2_ref, b2_ref, out_ref):
    wv = acc_ref[0, :N] + acc_ref[1, :N]
    zsum = (acc_ref[0, NPAD:NPAD + ZROWS]
            + acc_ref[1, NPAD:NPAD + ZROWS])
    kk = lax.broadcasted_iota(jnp.int32, (HID, 8 * HID), 0)
    mm = lax.broadcasted_iota(jnp.int32, (HID, 8 * HID), 1)
    sel = (kk == (mm // HID) * 16 + (mm % HID) // HD).astype(jnp.float32)
    zexp = jnp.dot(zsum, sel, preferred_element_type=jnp.float32)
    zfull = zexp.reshape(NPAD, HID)[:N]
    attn = wv / (zfull + 1e-6)
    h2 = h_ref[...] + jnp.dot(attn, wo_ref[...], preferred_element_type=jnp.float32) + bo_ref[...]
    h2 = _ln(h2, g1_ref[...], b1_ref[...])
    f = jnp.maximum(jnp.dot(h2, w1_ref[...], preferred_element_type=jnp.float32) + b1f_ref[...], 0.0)
    h2b = h2 + jnp.dot(f, w2_ref[...], preferred_element_type=jnp.float32) + b2f_ref[...]
    out_ref[...] = _ln(h2b, g2_ref[...], b2_ref[...])


_embed = pl.pallas_call(_embed_body, out_shape=jax.ShapeDtypeStruct((N, HID), jnp.float32))
_qkv = pl.pallas_call(
    _qkv_body,
    out_shape=(jax.ShapeDtypeStruct((N, 2 * HID), jnp.float32),
               jax.ShapeDtypeStruct((N, HID), jnp.float32)))
_post = pl.pallas_call(_post_body, out_shape=jax.ShapeDtypeStruct((N, HID), jnp.float32))


def kernel(x, edge_index, params):
    pad = EPW - E // NW
    src_r = jnp.pad(edge_index[0].reshape(NW, E // NW), ((0, 0), (0, pad))
                    ).reshape(NW, NBLK, BLK, C)
    dst_r = jnp.pad(edge_index[1].reshape(NW, E // NW), ((0, 0), (0, pad)),
                    constant_values=PAD_DST).reshape(NW, NBLK, BLK, C)
    h = _embed(x, params['We'].T)
    for p in params['layers']:
        wkv_t = jnp.concatenate([p['Wk'], p['Wv']], axis=0).T
        bkv = jnp.concatenate([p['bk'], p['bv']])[None, :]
        kv, q = _qkv(h, wkv_t, bkv, p['Wq'].T, p['bq'][None, :])
        acc2 = _edge_attn(kv, q, src_r, dst_r)
        h = _post(h, acc2, p['Wo'].T, p['bo'][None, :],
                  p['ln1_g'][None, :], p['ln1_b'][None, :],
                  p['W1'].T, p['b1f'][None, :], p['W2'].T, p['b2f'][None, :],
                  p['ln2_g'][None, :], p['ln2_b'][None, :])
    return h


# speedup vs baseline: 38.9729x; 2.3467x over previous
import jax
import jax.numpy as jnp
from jax import lax
from jax.experimental import pallas as pl
from jax.experimental.pallas import tpu as pltpu
from jax.experimental.pallas import tpu_sc as plsc

N = 10000
E = 320000
HID = 128
HEADS = 8
HD = 16

NC = 2
NS = 16
NW = NC * NS
C = 32
BLK = 8
NBLK = 40
CHUNKS = BLK * NBLK
EPW = C * CHUNKS
PAD_DST = N
NPAD = 10008
ZROWS = NPAD // 8
AROWS = 11264
APS = AROWS // NS

_mesh = plsc.VectorSubcoreMesh(core_axis_name="c", subcore_axis_name="s",
                               num_cores=NC, num_subcores=NS)

_GATHER_DNUMS = lax.GatherDimensionNumbers(
    offset_dims=(), collapsed_slice_dims=(0,), start_index_map=(0,))


def _lane_shuffle(v, idx):
    return lax.gather(v, idx[:, None], _GATHER_DNUMS, slice_sizes=(1,),
                      mode=lax.GatherScatterMode.PROMISE_IN_BOUNDS)


def _edge_body(kv_hbm, q_hbm, src_hbm, dst_hbm, acc_hbm,
               srcblk, dstblk, sidx, kv_v, q_v, o2,
               acc, sem1, sem2, sem3, sem4):
    c = lax.axis_index("c")
    s = lax.axis_index("s")
    w = s * NC + c

    zero16 = jnp.zeros((16,), jnp.float32)
    lanes = lax.iota(jnp.int32, 16)

    def zero_row(j, carry):
        for col in range(HID // 16):
            o2[j, pl.ds(col * 16, 16)] = zero16
        return carry
    lax.fori_loop(0, 2 * C, zero_row, 0)
    for t in range(APS // (2 * C)):
        pltpu.sync_copy(o2, acc.at[pl.ds(s * APS + t * 2 * C, 2 * C)])
    plsc.subcore_barrier()

    pltpu.sync_copy(src_hbm.at[w, 0], srcblk.at[0])
    pltpu.sync_copy(dst_hbm.at[w, 0], dstblk.at[0])
    pltpu.async_copy(kv_hbm.at[srcblk.at[0, 0]], kv_v.at[0], sem1)
    pltpu.async_copy(q_hbm.at[dstblk.at[0, 0]], q_v.at[0], sem2)

    def chunk(i, carry):
        p = i & 1
        b = lax.shift_right_logical(i, 3)
        j = i & (BLK - 1)
        pb = b & 1

        @pl.when((j == 0) & (b + 1 < NBLK))
        def _stage_next_block():
            pltpu.async_copy(src_hbm.at[w, b + 1], srcblk.at[1 - pb], sem3)
            pltpu.async_copy(dst_hbm.at[w, b + 1], dstblk.at[1 - pb], sem4)

        pltpu.make_async_copy(kv_hbm.at[pl.ds(0, C)], kv_v.at[p], sem1).wait()
        pltpu.make_async_copy(q_hbm.at[pl.ds(0, C)], q_v.at[p], sem2).wait()

        nj = (i + 1) & (BLK - 1)
        npb = lax.shift_right_logical(i + 1, 3) & 1

        @pl.when(j == BLK - 1)
        def _wait_next_block():
            @pl.when(b + 1 < NBLK)
            def _():
                pltpu.make_async_copy(src_hbm.at[w, 0], srcblk.at[0], sem3).wait()
                pltpu.make_async_copy(dst_hbm.at[w, 0], dstblk.at[0], sem4).wait()

        @pl.when(i + 1 < CHUNKS)
        def _issue_next_gather():
            pltpu.async_copy(kv_hbm.at[srcblk.at[npb, nj]], kv_v.at[1 - p], sem1)
            pltpu.async_copy(q_hbm.at[dstblk.at[npb, nj]], q_v.at[1 - p], sem2)

        for g in range(C // 16):
            dvec = dstblk[pb, j, pl.ds(g * 16, 16)]
            sidx[pl.ds(g * 16, 16)] = dvec
            sidx[pl.ds(C + g * 16, 16)] = (
                lax.shift_right_logical(dvec, 3) + NPAD)

        @plsc.parallel_loop(0, C, unroll=4)
        def edge(e):
            zvec = jnp.zeros((16,), jnp.float32)
            for h in range(HEADS):
                kh = kv_v[p, e, pl.ds(h * HD, 16)]
                qh = q_v[p, e, pl.ds(h * HD, 16)]
                sc = kh * qh
                for sh in (8, 4, 2, 1):
                    sc = sc + _lane_shuffle(sc, lanes ^ sh)
                sv = jnp.exp(jnp.clip(sc, -5.0, 5.0))
                vh = kv_v[p, e, pl.ds(HID + h * HD, 16)]
                o2[e, pl.ds(h * HD, 16)] = vh * sv
                zvec = jnp.where(lanes == h, sv, zvec)
            g = (e // 16) * 16
            dvec = dstblk[pb, j, pl.ds(g, 16)]
            db = _lane_shuffle(dvec, jnp.full((16,), e - g, jnp.int32))
            dmod = db & 7
            for off in range(8):
                m = 1.0 - jnp.minimum(jnp.abs(dmod - off), 1).astype(jnp.float32)
                o2[C + e, pl.ds(off * 16, 16)] = zvec * m

        pltpu.sync_copy(o2, acc.at[sidx], add=True)
        return carry
    lax.fori_loop(0, CHUNKS, chunk, 0)

    plsc.subcore_barrier()
    pltpu.sync_copy(acc.at[pl.ds(s * APS, APS)], acc_hbm.at[c, pl.ds(s * APS, APS)])


_edge_attn = pl.kernel(
    _edge_body,
    out_type=jax.ShapeDtypeStruct((NC, AROWS, HID), jnp.float32),
    mesh=_mesh,
    scratch_types=[
        pltpu.VMEM((2, BLK, C), jnp.int32),
        pltpu.VMEM((2, BLK, C), jnp.int32),
        pltpu.VMEM((2 * C,), jnp.int32),
        pltpu.VMEM((2, C, 2 * HID), jnp.float32),
        pltpu.VMEM((2, C, HID), jnp.float32),
        pltpu.VMEM((2 * C, HID), jnp.float32),
        pltpu.VMEM_SHARED((AROWS, HID), jnp.float32),
        pltpu.SemaphoreType.DMA,
        pltpu.SemaphoreType.DMA,
        pltpu.SemaphoreType.DMA,
        pltpu.SemaphoreType.DMA,
    ],
)


def _embed_body(x_ref, w_ref, o_ref):
    o_ref[...] = jnp.dot(x_ref[...], w_ref[...], preferred_element_type=jnp.float32)


def _qkv_body(h_ref, wkv_ref, bkv_ref, wq_ref, bq_ref, kv_ref, q_ref):
    hh = h_ref[...]
    kv_ref[...] = jnp.dot(hh, wkv_ref[...], preferred_element_type=jnp.float32) + bkv_ref[...]
    q_ref[...] = (jnp.dot(hh, wq_ref[...], preferred_element_type=jnp.float32)
                  + bq_ref[...]) * (HD ** -0.5)


def _ln(hh, g, b):
    mu = jnp.mean(hh, axis=-1, keepdims=True)
    var = jnp.mean((hh - mu) ** 2, axis=-1, keepdims=True)
    return (hh - mu) / jnp.sqrt(var + 1e-5) * g + b


def _post_body(h_ref, acc_ref, wo_ref, bo_ref, g1_ref, b1_ref,
               w1_ref, b1f_ref, w2_ref, b2f_ref, g2_ref, b2_ref, out_ref):
    wv = acc_ref[0, :N] + acc_ref[1, :N]
    zsum = (acc_ref[0, NPAD:NPAD + ZROWS]
            + acc_ref[1, NPAD:NPAD + ZROWS])
    kk = lax.broadcasted_iota(jnp.int32, (HID, 8 * HID), 0)
    mm = lax.broadcasted_iota(jnp.int32, (HID, 8 * HID), 1)
    sel = (kk == (mm // HID) * 16 + (mm % HID) // HD).astype(jnp.float32)
    zexp = jnp.dot(zsum, sel, preferred_element_type=jnp.float32)
    zfull = zexp.reshape(NPAD, HID)[:N]
    attn = wv / (zfull + 1e-6)
    h2 = h_ref[...] + jnp.dot(attn, wo_ref[...], preferred_element_type=jnp.float32) + bo_ref[...]
    h2 = _ln(h2, g1_ref[...], b1_ref[...])
    f = jnp.maximum(jnp.dot(h2, w1_ref[...], preferred_element_type=jnp.float32) + b1f_ref[...], 0.0)
    h2b = h2 + jnp.dot(f, w2_ref[...], preferred_element_type=jnp.float32) + b2f_ref[...]
    out_ref[...] = _ln(h2b, g2_ref[...], b2_ref[...])


_embed = pl.pallas_call(_embed_body, out_shape=jax.ShapeDtypeStruct((N, HID), jnp.float32))
_qkv = pl.pallas_call(
    _qkv_body,
    out_shape=(jax.ShapeDtypeStruct((N, 2 * HID), jnp.float32),
               jax.ShapeDtypeStruct((N, HID), jnp.float32)))
_post = pl.pallas_call(_post_body, out_shape=jax.ShapeDtypeStruct((N, HID), jnp.float32))


def kernel(x, edge_index, params):
    pad = EPW - E // NW
    src_r = jnp.pad(edge_index[0].reshape(NW, E // NW), ((0, 0), (0, pad))
                    ).reshape(NW, NBLK, BLK, C)
    dst_r = jnp.pad(edge_index[1].reshape(NW, E // NW), ((0, 0), (0, pad)),
                    constant_values=PAD_DST).reshape(NW, NBLK, BLK, C)
    h = _embed(x, params['We'].T)
    for p in params['layers']:
        wkv_t = jnp.concatenate([p['Wk'], p['Wv']], axis=0).T
        bkv = jnp.concatenate([p['bk'], p['bv']])[None, :]
        kv, q = _qkv(h, wkv_t, bkv, p['Wq'].T, p['bq'][None, :])
        acc2 = _edge_attn(kv, q, src_r, dst_r)
        h = _post(h, acc2, p['Wo'].T, p['bo'][None, :],
                  p['ln1_g'][None, :], p['ln1_b'][None, :],
                  p['W1'].T, p['b1f'][None, :], p['W2'].T, p['b2f'][None, :],
                  p['ln2_g'][None, :], p['ln2_b'][None, :])
    return h


# speedup vs baseline: 39.7067x; 1.0188x over previous
import jax
import jax.numpy as jnp
from jax import lax
from jax.experimental import pallas as pl
from jax.experimental.pallas import tpu as pltpu
from jax.experimental.pallas import tpu_sc as plsc

N = 10000
E = 320000
HID = 128
HEADS = 8
HD = 16

NC = 2
NS = 16
NW = NC * NS
C = 32
BLK = 8
NBLK = 40
CHUNKS = BLK * NBLK
EPW = C * CHUNKS
PAD_DST = N
NPAD = 10008
ZROWS = NPAD // 8
AROWS = 11264
APS = AROWS // NS

_mesh = plsc.VectorSubcoreMesh(core_axis_name="c", subcore_axis_name="s",
                               num_cores=NC, num_subcores=NS)

_GATHER_DNUMS = lax.GatherDimensionNumbers(
    offset_dims=(), collapsed_slice_dims=(0,), start_index_map=(0,))


def _lane_shuffle(v, idx):
    return lax.gather(v, idx[:, None], _GATHER_DNUMS, slice_sizes=(1,),
                      mode=lax.GatherScatterMode.PROMISE_IN_BOUNDS)


def _edge_body(kv_hbm, q_hbm, src_hbm, dst_hbm, acc_hbm,
               srcblk, dstblk, sidx, kv_v, q_v, o2,
               acc, sem1, sem2, sem3, sem4, sem5):
    c = lax.axis_index("c")
    s = lax.axis_index("s")
    w = s * NC + c

    zero16 = jnp.zeros((16,), jnp.float32)
    lanes = lax.iota(jnp.int32, 16)

    def zero_row(j, carry):
        for col in range(HID // 16):
            o2[j, pl.ds(col * 16, 16)] = zero16
        return carry
    lax.fori_loop(0, 2 * C, zero_row, 0)
    for t in range(APS // (2 * C)):
        pltpu.sync_copy(o2, acc.at[pl.ds(s * APS + t * 2 * C, 2 * C)])
    plsc.subcore_barrier()

    pltpu.sync_copy(src_hbm.at[w, 0], srcblk.at[0])
    pltpu.sync_copy(dst_hbm.at[w, 0], dstblk.at[0])
    pltpu.async_copy(kv_hbm.at[srcblk.at[0, 0]], kv_v.at[0], sem1)
    pltpu.async_copy(q_hbm.at[dstblk.at[0, 0]], q_v.at[0], sem2)

    def chunk(i, carry):
        p = i & 1
        b = lax.shift_right_logical(i, 3)
        j = i & (BLK - 1)
        pb = b & 1

        @pl.when((j == 0) & (b + 1 < NBLK))
        def _stage_next_block():
            pltpu.async_copy(src_hbm.at[w, b + 1], srcblk.at[1 - pb], sem3)
            pltpu.async_copy(dst_hbm.at[w, b + 1], dstblk.at[1 - pb], sem4)

        pltpu.make_async_copy(kv_hbm.at[pl.ds(0, C)], kv_v.at[p], sem1).wait()
        pltpu.make_async_copy(q_hbm.at[pl.ds(0, C)], q_v.at[p], sem2).wait()

        nj = (i + 1) & (BLK - 1)
        npb = lax.shift_right_logical(i + 1, 3) & 1

        @pl.when(j == BLK - 1)
        def _wait_next_block():
            @pl.when(b + 1 < NBLK)
            def _():
                pltpu.make_async_copy(src_hbm.at[w, 0], srcblk.at[0], sem3).wait()
                pltpu.make_async_copy(dst_hbm.at[w, 0], dstblk.at[0], sem4).wait()

        @pl.when(i + 1 < CHUNKS)
        def _issue_next_gather():
            pltpu.async_copy(kv_hbm.at[srcblk.at[npb, nj]], kv_v.at[1 - p], sem1)
            pltpu.async_copy(q_hbm.at[dstblk.at[npb, nj]], q_v.at[1 - p], sem2)

        for g in range(C // 16):
            dvec = dstblk[pb, j, pl.ds(g * 16, 16)]
            sidx[p, pl.ds(g * 16, 16)] = dvec
            sidx[p, pl.ds(C + g * 16, 16)] = (
                lax.shift_right_logical(dvec, 3) + NPAD)

        @pl.when(i > 0)
        def _wait_prev_scatter():
            pltpu.make_async_copy(o2, acc.at[sidx.at[1 - p]], sem5).wait()

        @plsc.parallel_loop(0, C, unroll=4)
        def edge(e):
            zvec = jnp.zeros((16,), jnp.float32)
            for h in range(HEADS):
                kh = kv_v[p, e, pl.ds(h * HD, 16)]
                qh = q_v[p, e, pl.ds(h * HD, 16)]
                sc = kh * qh
                for sh in (8, 4, 2, 1):
                    sc = sc + _lane_shuffle(sc, lanes ^ sh)
                sv = jnp.exp(jnp.clip(sc, -5.0, 5.0))
                vh = kv_v[p, e, pl.ds(HID + h * HD, 16)]
                o2[e, pl.ds(h * HD, 16)] = vh * sv
                zvec = jnp.where(lanes == h, sv, zvec)
            g = (e // 16) * 16
            dvec = dstblk[pb, j, pl.ds(g, 16)]
            db = _lane_shuffle(dvec, jnp.full((16,), e - g, jnp.int32))
            dmod = db & 7
            for off in range(8):
                m = 1.0 - jnp.minimum(jnp.abs(dmod - off), 1).astype(jnp.float32)
                o2[C + e, pl.ds(off * 16, 16)] = zvec * m

        pltpu.async_copy(o2, acc.at[sidx.at[p]], sem5, add=True)
        return carry
    lax.fori_loop(0, CHUNKS, chunk, 0)

    pltpu.make_async_copy(o2, acc.at[sidx.at[(CHUNKS - 1) & 1]], sem5).wait()
    plsc.subcore_barrier()
    pltpu.sync_copy(acc.at[pl.ds(s * APS, APS)], acc_hbm.at[c, pl.ds(s * APS, APS)])


_edge_attn = pl.kernel(
    _edge_body,
    out_type=jax.ShapeDtypeStruct((NC, AROWS, HID), jnp.float32),
    mesh=_mesh,
    scratch_types=[
        pltpu.VMEM((2, BLK, C), jnp.int32),
        pltpu.VMEM((2, BLK, C), jnp.int32),
        pltpu.VMEM((2, 2 * C), jnp.int32),
        pltpu.VMEM((2, C, 2 * HID), jnp.float32),
        pltpu.VMEM((2, C, HID), jnp.float32),
        pltpu.VMEM((2 * C, HID), jnp.float32),
        pltpu.VMEM_SHARED((AROWS, HID), jnp.float32),
        pltpu.SemaphoreType.DMA,
        pltpu.SemaphoreType.DMA,
        pltpu.SemaphoreType.DMA,
        pltpu.SemaphoreType.DMA,
        pltpu.SemaphoreType.DMA,
    ],
)


def _embed_body(x_ref, w_ref, o_ref):
    o_ref[...] = jnp.dot(x_ref[...], w_ref[...], preferred_element_type=jnp.float32)


def _qkv_body(h_ref, wkv_ref, bkv_ref, wq_ref, bq_ref, kv_ref, q_ref):
    hh = h_ref[...]
    kv_ref[...] = jnp.dot(hh, wkv_ref[...], preferred_element_type=jnp.float32) + bkv_ref[...]
    q_ref[...] = (jnp.dot(hh, wq_ref[...], preferred_element_type=jnp.float32)
                  + bq_ref[...]) * (HD ** -0.5)


def _ln(hh, g, b):
    mu = jnp.mean(hh, axis=-1, keepdims=True)
    var = jnp.mean((hh - mu) ** 2, axis=-1, keepdims=True)
    return (hh - mu) / jnp.sqrt(var + 1e-5) * g + b


def _post_body(h_ref, acc_ref, wo_ref, bo_ref, g1_ref, b1_ref,
               w1_ref, b1f_ref, w2_ref, b2f_ref, g2_ref, b2_ref, out_ref):
    wv = acc_ref[0, :N] + acc_ref[1, :N]
    zsum = (acc_ref[0, NPAD:NPAD + ZROWS]
            + acc_ref[1, NPAD:NPAD + ZROWS])
    kk = lax.broadcasted_iota(jnp.int32, (HID, 8 * HID), 0)
    mm = lax.broadcasted_iota(jnp.int32, (HID, 8 * HID), 1)
    sel = (kk == (mm // HID) * 16 + (mm % HID) // HD).astype(jnp.float32)
    zexp = jnp.dot(zsum, sel, preferred_element_type=jnp.float32)
    zfull = zexp.reshape(NPAD, HID)[:N]
    attn = wv / (zfull + 1e-6)
    h2 = h_ref[...] + jnp.dot(attn, wo_ref[...], preferred_element_type=jnp.float32) + bo_ref[...]
    h2 = _ln(h2, g1_ref[...], b1_ref[...])
    f = jnp.maximum(jnp.dot(h2, w1_ref[...], preferred_element_type=jnp.float32) + b1f_ref[...], 0.0)
    h2b = h2 + jnp.dot(f, w2_ref[...], preferred_element_type=jnp.float32) + b2f_ref[...]
    out_ref[...] = _ln(h2b, g2_ref[...], b2_ref[...])


_embed = pl.pallas_call(_embed_body, out_shape=jax.ShapeDtypeStruct((N, HID), jnp.float32))
_qkv = pl.pallas_call(
    _qkv_body,
    out_shape=(jax.ShapeDtypeStruct((N, 2 * HID), jnp.float32),
               jax.ShapeDtypeStruct((N, HID), jnp.float32)))
_post = pl.pallas_call(_post_body, out_shape=jax.ShapeDtypeStruct((N, HID), jnp.float32))


def kernel(x, edge_index, params):
    pad = EPW - E // NW
    src_r = jnp.pad(edge_index[0].reshape(NW, E // NW), ((0, 0), (0, pad))
                    ).reshape(NW, NBLK, BLK, C)
    dst_r = jnp.pad(edge_index[1].reshape(NW, E // NW), ((0, 0), (0, pad)),
                    constant_values=PAD_DST).reshape(NW, NBLK, BLK, C)
    h = _embed(x, params['We'].T)
    for p in params['layers']:
        wkv_t = jnp.concatenate([p['Wk'], p['Wv']], axis=0).T
        bkv = jnp.concatenate([p['bk'], p['bv']])[None, :]
        kv, q = _qkv(h, wkv_t, bkv, p['Wq'].T, p['bq'][None, :])
        acc2 = _edge_attn(kv, q, src_r, dst_r)
        h = _post(h, acc2, p['Wo'].T, p['bo'][None, :],
                  p['ln1_g'][None, :], p['ln1_b'][None, :],
                  p['W1'].T, p['b1f'][None, :], p['W2'].T, p['b2f'][None, :],
                  p['ln2_g'][None, :], p['ln2_b'][None, :])
    return h
